# baseline (speedup 1.0000x reference)
import functools

import jax
import jax.numpy as jnp
from jax import lax
from jax.experimental import pallas as pl
from jax.experimental.pallas import tpu as pltpu
from jax.experimental.pallas import tpu_sc as plsc


def kernel(x, pos_table):
    B, S, D = x.shape
    dtype = pos_table.dtype

    info = plsc.get_sparse_core_info()
    NC, NS = info.num_cores, info.num_subcores
    NW = NC * NS
    rows_per_w = S // NW

    mesh = plsc.VectorSubcoreMesh(core_axis_name="c", subcore_axis_name="s")

    @functools.partial(
        pl.kernel,
        mesh=mesh,
        out_type=jax.ShapeDtypeStruct((B, S, D), dtype),
        scratch_types=[
            pltpu.SemaphoreType.DMA,
        ],
    )
    def broadcast_rows(table_hbm, out_hbm, sem):
        wid = lax.axis_index("s") * NC + lax.axis_index("c")
        row0 = wid * rows_per_w
        copies = [
            pltpu.async_copy(
                table_hbm.at[pl.ds(row0, rows_per_w)],
                out_hbm.at[b, pl.ds(row0, rows_per_w)],
                sem,
            )
            for b in range(B)
        ]
        for h in copies:
            h.wait()

    return broadcast_rows(pos_table)


# speedup vs baseline: 45.4723x; 45.4723x over previous
import functools

import jax
import jax.numpy as jnp
from jax import lax
from jax.experimental import pallas as pl
from jax.experimental.pallas import tpu as pltpu
from jax.experimental.pallas import tpu_sc as plsc


def kernel(x, pos_table):
    B, S, D = x.shape
    dtype = pos_table.dtype

    info = plsc.get_sparse_core_info()
    NC, NS = info.num_cores, info.num_subcores
    NW = NC * NS
    rows_per_w = S // NW
    BIG = 56
    chunk_rows = [BIG, BIG, rows_per_w - 2 * BIG]
    chunk_off = [0, BIG, 2 * BIG]
    nchunks = len(chunk_rows)
    NBUF = 2

    mesh = plsc.VectorSubcoreMesh(core_axis_name="c", subcore_axis_name="s")

    @functools.partial(
        pl.kernel,
        mesh=mesh,
        out_type=jax.ShapeDtypeStruct((B, S, D), dtype),
        scratch_types=[
            pltpu.VMEM((NBUF, BIG, D), dtype),
            pltpu.SemaphoreType.DMA,
            pltpu.SemaphoreType.DMA,
        ],
    )
    def broadcast_rows(table_hbm, out_hbm, buf, lsem, ssem):
        wid = lax.axis_index("s") * NC + lax.axis_index("c")
        row0 = wid * rows_per_w

        loads = [None] * nchunks
        stores = [None] * nchunks

        def start_load(c):
            loads[c] = pltpu.async_copy(
                table_hbm.at[pl.ds(row0 + chunk_off[c], chunk_rows[c])],
                buf.at[c % NBUF, pl.ds(0, chunk_rows[c])],
                lsem,
            )

        for n in range(min(NBUF, nchunks)):
            start_load(n)
        for c in range(nchunks):
            if c >= 1:
                for h in stores[c - 1]:
                    h.wait()
                n = (c - 1) + NBUF
                if n < nchunks:
                    start_load(n)
            loads[c].wait()
            stores[c] = [
                pltpu.async_copy(
                    buf.at[c % NBUF, pl.ds(0, chunk_rows[c])],
                    out_hbm.at[b, pl.ds(row0 + chunk_off[c], chunk_rows[c])],
                    ssem,
                )
                for b in range(B)
            ]
        for h in stores[nchunks - 1]:
            h.wait()

    return broadcast_rows(pos_table)


# speedup vs baseline: 45.4984x; 1.0006x over previous
import functools

import jax
import jax.numpy as jnp
from jax import lax
from jax.experimental import pallas as pl
from jax.experimental.pallas import tpu as pltpu
from jax.experimental.pallas import tpu_sc as plsc


def kernel(x, pos_table):
    B, S, D = x.shape
    dtype = pos_table.dtype

    info = plsc.get_sparse_core_info()
    NC, NS = info.num_cores, info.num_subcores
    NW = NC * NS
    rows_per_w = S // NW
    BIG = 56
    chunk_rows = [BIG, BIG, rows_per_w - 2 * BIG]
    chunk_off = [0, BIG, 2 * BIG]
    nchunks = len(chunk_rows)
    NBUF = 2

    mesh = plsc.VectorSubcoreMesh(core_axis_name="c", subcore_axis_name="s")

    @functools.partial(
        pl.kernel,
        mesh=mesh,
        out_type=jax.ShapeDtypeStruct((B, S, D), dtype),
        scratch_types=[
            pltpu.VMEM((NBUF, BIG, D), dtype),
            pltpu.SemaphoreType.DMA,
            pltpu.SemaphoreType.DMA,
        ],
    )
    def broadcast_rows(table_hbm, out_hbm, buf, lsem, ssem):
        wid = lax.axis_index("s") * NC + lax.axis_index("c")
        row0 = wid * rows_per_w

        loads = [None] * nchunks
        stores = [None] * nchunks

        def start_load(c):
            loads[c] = pltpu.async_copy(
                table_hbm.at[pl.ds(row0 + chunk_off[c], chunk_rows[c])],
                buf.at[c % NBUF, pl.ds(0, chunk_rows[c])],
                lsem,
            )

        for n in range(min(NBUF, nchunks)):
            start_load(n)
        next_load = NBUF
        drained = [False] * nchunks
        for c in range(nchunks):
            loads[c].wait()
            stores[c] = [
                pltpu.async_copy(
                    buf.at[c % NBUF, pl.ds(0, chunk_rows[c])],
                    out_hbm.at[b, pl.ds(row0 + chunk_off[c], chunk_rows[c])],
                    ssem,
                )
                for b in range(B)
            ]
            if next_load < nchunks and c == next_load - 1:
                for h in stores[next_load - NBUF]:
                    h.wait()
                drained[next_load - NBUF] = True
                start_load(next_load)
                next_load += 1
        for c in range(nchunks):
            if not drained[c]:
                for h in stores[c]:
                    h.wait()

    return broadcast_rows(pos_table)
